# Initial kernel scaffold; baseline (speedup 1.0000x reference)
import jax
import jax.numpy as jnp
from jax.experimental import pallas as pl


def kernel(X, edge_index, W1, b1, W2, b2, Wc, bc):
    raise NotImplementedError("write your pallas kernel here")


# speedup vs baseline: 12.7498x; 12.7498x over previous
import functools

import jax
import jax.numpy as jnp
from jax import lax
from jax.experimental import pallas as pl
from jax.experimental.pallas import tpu as pltpu
from jax.experimental.pallas import tpu_sc as plsc

N = 10000
E = 320000
NC = 2
NS = 16
NW = NC * NS
CH = 80
EPW = E // NW
NCHUNK = EPW // CH
NPAD = 10240
RPT = NPAD // NS
DEGW = 16

_mesh = plsc.VectorSubcoreMesh(core_axis_name="c", subcore_axis_name="s")


@functools.partial(
    pl.kernel,
    out_type=jax.ShapeDtypeStruct((NC, NPAD, DEGW), jnp.float32),
    mesh=_mesh,
    scratch_types=[
        pltpu.VMEM((CH,), jnp.int32),
        pltpu.VMEM((CH, DEGW), jnp.float32),
        pltpu.VMEM_SHARED((NPAD, DEGW), jnp.float32),
    ],
    compiler_params=pltpu.CompilerParams(use_tc_tiling_on_sc=False),
)
def _sc_degree(dst_hbm, ones_hbm, zeros_hbm, out_hbm, dst_v, ones_v, acc):
    c = lax.axis_index("c")
    s = lax.axis_index("s")
    r0 = s * RPT
    pltpu.sync_copy(ones_hbm, ones_v)
    pltpu.sync_copy(zeros_hbm.at[pl.ds(r0, RPT)], acc.at[pl.ds(r0, RPT)])
    plsc.subcore_barrier()
    base = (c * NS + s) * EPW

    def body(i, carry):
        off = base + i * CH
        pltpu.sync_copy(dst_hbm.at[pl.ds(off, CH)], dst_v)
        pltpu.sync_copy(ones_v, acc.at[dst_v], add=True)
        return carry

    lax.fori_loop(0, NCHUNK, body, 0)
    plsc.subcore_barrier()
    pltpu.sync_copy(acc.at[pl.ds(r0, RPT)], out_hbm.at[c, pl.ds(r0, RPT)])


def _make_sc_prop(D):
    @functools.partial(
        pl.kernel,
        out_type=jax.ShapeDtypeStruct((NC, NPAD, D), jnp.float32),
        mesh=_mesh,
        scratch_types=[
            pltpu.VMEM((CH,), jnp.int32),
            pltpu.VMEM((CH,), jnp.int32),
            pltpu.VMEM((CH, D), jnp.float32),
            pltpu.VMEM_SHARED((NPAD, D), jnp.float32),
            pltpu.SemaphoreType.DMA,
        ],
        compiler_params=pltpu.CompilerParams(
            use_tc_tiling_on_sc=(D % 128 == 0)),
    )
    def prop(g_hbm, src_hbm, dst_hbm, zeros_hbm, out_hbm,
             src_v, dst_v, rows_v, acc, sem):
        c = lax.axis_index("c")
        s = lax.axis_index("s")
        r0 = s * RPT
        pltpu.sync_copy(zeros_hbm.at[pl.ds(r0, RPT)], acc.at[pl.ds(r0, RPT)])
        plsc.subcore_barrier()
        base = (c * NS + s) * EPW

        def body(i, carry):
            off = base + i * CH
            pltpu.sync_copy(src_hbm.at[pl.ds(off, CH)], src_v)
            pltpu.sync_copy(dst_hbm.at[pl.ds(off, CH)], dst_v)
            pltpu.async_copy(g_hbm.at[src_v], rows_v, sem).wait()
            pltpu.sync_copy(rows_v, acc.at[dst_v], add=True)
            return carry

        lax.fori_loop(0, NCHUNK, body, 0)
        plsc.subcore_barrier()
        pltpu.sync_copy(acc.at[pl.ds(r0, RPT)], out_hbm.at[c, pl.ds(r0, RPT)])

    return prop


_sc_prop128 = _make_sc_prop(128)
_sc_prop64 = _make_sc_prop(64)


BR = 400


def _dinv(d0, d1):
    deg = d0[:, :1] + d1[:, :1] + 1.0
    return lax.rsqrt(deg)


def _stage_a_body(x_ref, w1_ref, d0_ref, d1_ref, g_ref):
    dinv = _dinv(d0_ref[...], d1_ref[...])
    h = jnp.dot(x_ref[...], w1_ref[...], preferred_element_type=jnp.float32)
    g_ref[...] = h * dinv


def _stage_b_body(s0_ref, s1_ref, g1_ref, d0_ref, d1_ref, b1_ref, w2_ref,
                  g2_ref):
    dinv = _dinv(d0_ref[...], d1_ref[...])
    t = s0_ref[...] + s1_ref[...] + g1_ref[...]
    out1 = jnp.maximum(t * dinv + b1_ref[...], 0.0)
    h2 = jnp.dot(out1, w2_ref[...], preferred_element_type=jnp.float32)
    g2_ref[...] = h2 * dinv


def _stage_c_body(s0_ref, s1_ref, g2_ref, d0_ref, d1_ref, b2_ref, wc_ref,
                  bc_ref, out_ref):
    dinv = _dinv(d0_ref[...], d1_ref[...])
    t = s0_ref[...] + s1_ref[...] + g2_ref[...]
    h2 = jnp.maximum(t * dinv + b2_ref[...], 0.0)
    out_ref[...] = (
        jnp.dot(h2, wc_ref[...], preferred_element_type=jnp.float32)
        + bc_ref[...])


def _row_spec(w):
    return pl.BlockSpec((BR, w), lambda i: (i, 0))


def _full_spec(r, w):
    return pl.BlockSpec((r, w), lambda i: (0, 0))


def _stage_a(X, W1, d0, d1):
    return pl.pallas_call(
        _stage_a_body,
        grid=(N // BR,),
        in_specs=[_row_spec(128), _full_spec(128, 128),
                  _row_spec(DEGW), _row_spec(DEGW)],
        out_specs=_row_spec(128),
        out_shape=jax.ShapeDtypeStruct((N, 128), jnp.float32),
    )(X, W1, d0, d1)


def _stage_b(s0, s1, g1, d0, d1, b1, W2):
    return pl.pallas_call(
        _stage_b_body,
        grid=(N // BR,),
        in_specs=[_row_spec(128), _row_spec(128), _row_spec(128),
                  _row_spec(DEGW), _row_spec(DEGW),
                  _full_spec(1, 128), _full_spec(128, 64)],
        out_specs=_row_spec(64),
        out_shape=jax.ShapeDtypeStruct((N, 64), jnp.float32),
    )(s0, s1, g1, d0, d1, b1, W2)


def _stage_c(s0, s1, g2, d0, d1, b2, Wc, bc):
    return pl.pallas_call(
        _stage_c_body,
        grid=(N // BR,),
        in_specs=[_row_spec(64), _row_spec(64), _row_spec(64),
                  _row_spec(DEGW), _row_spec(DEGW),
                  _full_spec(1, 64), _full_spec(64, 32), _full_spec(1, 32)],
        out_specs=_row_spec(32),
        out_shape=jax.ShapeDtypeStruct((N, 32), jnp.float32),
    )(s0, s1, g2, d0, d1, b2, Wc, bc)


def kernel(X, edge_index, W1, b1, W2, b2, Wc, bc):
    src = edge_index[0].astype(jnp.int32)
    dst = edge_index[1].astype(jnp.int32)

    ones_w = jnp.ones((CH, DEGW), jnp.float32)
    zeros_w = jnp.zeros((NPAD, DEGW), jnp.float32)
    zeros128 = jnp.zeros((NPAD, 128), jnp.float32)
    zeros64 = jnp.zeros((NPAD, 64), jnp.float32)

    degp = _sc_degree(dst, ones_w, zeros_w)
    d0, d1 = degp[0, :N], degp[1, :N]

    g1 = _stage_a(X, W1, d0, d1)
    g1p = jnp.pad(g1, ((0, NPAD - N), (0, 0)))
    s1p = _sc_prop128(g1p, src, dst, zeros128)
    s1 = s1p[:, :N]
    g2 = _stage_b(s1[0], s1[1], g1, d0, d1, b1.reshape(1, -1), W2)
    g2p = jnp.pad(g2, ((0, NPAD - N), (0, 0)))
    s2p = _sc_prop64(g2p, src, dst, zeros64)
    s2 = s2p[:, :N]
    return _stage_c(s2[0], s2[1], g2, d0, d1, b2.reshape(1, -1), Wc,
                    bc.reshape(1, -1))
